# Initial kernel scaffold; baseline (speedup 1.0000x reference)
import jax
import jax.numpy as jnp
from jax.experimental import pallas as pl


def kernel(H, V, E, rev_index):
    raise NotImplementedError("write your pallas kernel here")


# speedup vs baseline: 1.0203x; 1.0203x over previous
import functools

import jax
import jax.numpy as jnp
from jax import lax
from jax.experimental import pallas as pl
from jax.experimental.pallas import tpu as pltpu
from jax.experimental.pallas import tpu_sc as plsc

N_EDGES = 320000
D_FEAT = 128
LANES = 16
VREGS_PER_ROW = D_FEAT // LANES

_info = plsc.get_sparse_core_info()
NC = _info.num_cores
NS = _info.num_subcores
NW = NC * NS
ROWS_PER_W = N_EDGES // NW
CHUNK = 400
N_CHUNKS = ROWS_PER_W // CHUNK


def _sc_body(h_hbm, idx_hbm, out_hbm, idx_v, rows_v, seq_v, gsem):
    wid = lax.axis_index("s") * NC + lax.axis_index("c")
    base_w = wid * ROWS_PER_W

    def chunk_body(i, _):
        base = base_w + i * CHUNK
        pltpu.sync_copy(idx_hbm.at[pl.ds(base, CHUNK)], idx_v)
        gather = pltpu.async_copy(h_hbm.at[idx_v], rows_v, gsem)
        pltpu.sync_copy(h_hbm.at[pl.ds(base, CHUNK)], seq_v)
        gather.wait()

        def row_body(j, _):
            for l in range(VREGS_PER_ROW):
                sl = pl.ds(l * LANES, LANES)
                seq_v[j, sl] = (seq_v[j, sl] + rows_v[j, sl]) * 0.5
            return 0

        lax.fori_loop(0, CHUNK, row_body, 0)
        pltpu.sync_copy(seq_v, out_hbm.at[pl.ds(base, CHUNK)])
        return 0

    lax.fori_loop(0, N_CHUNKS, chunk_body, 0)


@jax.jit
def _message_sym(H, rev_index):
    mesh = plsc.VectorSubcoreMesh(core_axis_name="c", subcore_axis_name="s")
    fn = functools.partial(
        pl.kernel,
        mesh=mesh,
        out_type=jax.ShapeDtypeStruct((N_EDGES, D_FEAT), jnp.float32),
        scratch_types=[
            pltpu.VMEM((CHUNK,), jnp.int32),
            pltpu.VMEM((CHUNK, D_FEAT), jnp.float32),
            pltpu.VMEM((CHUNK, D_FEAT), jnp.float32),
            pltpu.SemaphoreType.DMA,
        ],
    )(_sc_body)
    return fn(H, rev_index)


def kernel(H, V, E, rev_index):
    return _message_sym(H, rev_index.astype(jnp.int32))
